# Initial kernel scaffold; baseline (speedup 1.0000x reference)
import jax
import jax.numpy as jnp
from jax.experimental import pallas as pl


def kernel(x, edge_index, batch, W1, b1, W2, b2, W3, b3, Wlin, blin):
    raise NotImplementedError("write your pallas kernel here")


# speedup vs baseline: 8.1060x; 8.1060x over previous
import functools

import jax
import jax.numpy as jnp
from jax import lax
from jax.experimental import pallas as pl
from jax.experimental.pallas import tpu as pltpu
from jax.experimental.pallas import tpu_sc as plsc

NN = 10000
DD = 128
HD = 64
GG = 64
NP = 10240
NW = 32
TILES = 16
ROWS_PER_TILE = NP // TILES
CHUNK = 128
EPW_CHUNKS = 80
EP = NW * EPW_CHUNKS * CHUNK
BR = 1000
GRID = NN // BR

@functools.cache
def _sc_kernels():
    mesh = plsc.VectorSubcoreMesh(
        core_axis_name="c", subcore_axis_name="s", num_cores=2,
        num_subcores=TILES,
    )

    @functools.partial(
        pl.kernel,
        out_type=jax.ShapeDtypeStruct((2, NP, 16), jnp.float32),
        mesh=mesh,
        compiler_params=pltpu.CompilerParams(use_tc_tiling_on_sc=False),
        scratch_types=[
            pltpu.VMEM((EPW_CHUNKS, CHUNK), jnp.int32),
            pltpu.VMEM((CHUNK, 16), jnp.float32),
            pltpu.VMEM((CHUNK, 16), jnp.float32),
            pltpu.VMEM_SHARED((NP, 16), jnp.float32),
            pltpu.SemaphoreType.DMA,
        ],
    )
    def deg_kernel(dstp, ones16, zeros16, degp, dst_v, ones_v, zb, acc, sem):
        c = lax.axis_index("c")
        s = lax.axis_index("s")
        w = c * TILES + s
        pltpu.sync_copy(dstp.at[w], dst_v)
        pltpu.sync_copy(ones16, ones_v)
        pltpu.sync_copy(zeros16, zb)
        r0 = s * ROWS_PER_TILE
        for k in range(ROWS_PER_TILE // CHUNK):
            pltpu.sync_copy(zb, acc.at[pl.ds(r0 + k * CHUNK, CHUNK)])
        plsc.subcore_barrier()

        def body(j, carry):
            pltpu.sync_copy(ones_v, acc.at[dst_v.at[j]], add=True)
            return carry

        lax.fori_loop(0, EPW_CHUNKS, body, 0)
        plsc.subcore_barrier()
        for k in range(ROWS_PER_TILE // CHUNK):
            pltpu.sync_copy(acc.at[pl.ds(r0 + k * CHUNK, CHUNK)], zb)
            pltpu.sync_copy(zb, degp.at[c, pl.ds(r0 + k * CHUNK, CHUNK)])

    @functools.partial(
        pl.kernel,
        out_type=jax.ShapeDtypeStruct((2, NP, HD), jnp.float32),
        mesh=mesh,
        compiler_params=pltpu.CompilerParams(use_tc_tiling_on_sc=False),
        scratch_types=[
            pltpu.VMEM((2 * EPW_CHUNKS, CHUNK), jnp.int32),
            pltpu.VMEM((2 * EPW_CHUNKS, CHUNK), jnp.int32),
            pltpu.VMEM((CHUNK, HD), jnp.float32),
            pltpu.VMEM((CHUNK, HD), jnp.float32),
            pltpu.VMEM_SHARED((NP, HD), jnp.float32),
            pltpu.SemaphoreType.DMA,
        ],
    )
    def prop_kernel(g2, srcev, srcod, dstp, zeros64, parts,
                    src_v, dst_v, rows, zb, acc, sem):
        c = lax.axis_index("c")
        s = lax.axis_index("s")

        @pl.when(c == 0)
        def _():
            pltpu.sync_copy(srcev.at[2 * s], src_v.at[pl.ds(0, EPW_CHUNKS)])
            pltpu.sync_copy(srcev.at[2 * s + 1],
                            src_v.at[pl.ds(EPW_CHUNKS, EPW_CHUNKS)])

        @pl.when(c == 1)
        def _():
            pltpu.sync_copy(srcod.at[2 * s], src_v.at[pl.ds(0, EPW_CHUNKS)])
            pltpu.sync_copy(srcod.at[2 * s + 1],
                            src_v.at[pl.ds(EPW_CHUNKS, EPW_CHUNKS)])

        pltpu.sync_copy(dstp.at[2 * s], dst_v.at[pl.ds(0, EPW_CHUNKS)])
        pltpu.sync_copy(dstp.at[2 * s + 1],
                        dst_v.at[pl.ds(EPW_CHUNKS, EPW_CHUNKS)])
        pltpu.sync_copy(zeros64, zb)
        r0 = s * ROWS_PER_TILE
        for k in range(ROWS_PER_TILE // CHUNK):
            pltpu.sync_copy(zb, acc.at[pl.ds(r0 + k * CHUNK, CHUNK)])
        plsc.subcore_barrier()

        def body(j, carry):
            pltpu.async_copy(g2.at[src_v.at[j]], rows, sem).wait()
            pltpu.sync_copy(rows, acc.at[dst_v.at[j]], add=True)
            return carry

        lax.fori_loop(0, 2 * EPW_CHUNKS, body, 0)
        plsc.subcore_barrier()
        for k in range(ROWS_PER_TILE // CHUNK):
            pltpu.sync_copy(acc.at[pl.ds(r0 + k * CHUNK, CHUNK)], zb)
            pltpu.sync_copy(zb, parts.at[c, pl.ds(r0 + k * CHUNK, CHUNK)])

    return deg_kernel, prop_kernel


def _run_deg(dstp, ones16, zeros16):
    return _sc_kernels()[0](dstp, ones16, zeros16)


def _run_prop(g, srcev, srcod, dstp, zeros64):
    g2 = g.reshape(2 * NN, HD)
    return _sc_kernels()[1](g2, srcev, srcod, dstp, zeros64)


def _tc0_body(x_ref, w_ref, degp_ref, g_ref, dinv_ref):
    dp = degp_ref[...]
    deg = dp[0, :, 0:1] + dp[1, :, 0:1] + 1.0
    dinv = 1.0 / jnp.sqrt(deg)
    h = jnp.dot(x_ref[...], w_ref[...], preferred_element_type=jnp.float32)
    g_ref[...] = h * dinv
    dinv_ref[...] = dinv


def _tc0(x, w1, degp):
    return pl.pallas_call(
        _tc0_body,
        grid=(GRID,),
        in_specs=[
            pl.BlockSpec((BR, DD), lambda i: (i, 0)),
            pl.BlockSpec((DD, DD), lambda i: (0, 0)),
            pl.BlockSpec((2, BR, 16), lambda i: (0, i, 0)),
        ],
        out_specs=[
            pl.BlockSpec((BR, DD), lambda i: (i, 0)),
            pl.BlockSpec((BR, 1), lambda i: (i, 0)),
        ],
        out_shape=[
            jax.ShapeDtypeStruct((NN, DD), jnp.float32),
            jax.ShapeDtypeStruct((NN, 1), jnp.float32),
        ],
    )(x, w1, degp)


def _tcmid_body(p_ref, g_ref, dinv_ref, b_ref, w_ref, out_ref):
    p = p_ref[...]
    s = jnp.concatenate([p[0], p[1]], axis=1)
    dinv = dinv_ref[...]
    xb = (s + g_ref[...]) * dinv + b_ref[...]
    xb = jnp.maximum(xb, 0.0)
    out_ref[...] = (
        jnp.dot(xb, w_ref[...], preferred_element_type=jnp.float32) * dinv
    )


def _tcmid(parts, gprev, dinv, b, wnext):
    return pl.pallas_call(
        _tcmid_body,
        grid=(GRID,),
        in_specs=[
            pl.BlockSpec((2, BR, HD), lambda i: (0, i, 0)),
            pl.BlockSpec((BR, DD), lambda i: (i, 0)),
            pl.BlockSpec((BR, 1), lambda i: (i, 0)),
            pl.BlockSpec((1, DD), lambda i: (0, 0)),
            pl.BlockSpec((DD, DD), lambda i: (0, 0)),
        ],
        out_specs=pl.BlockSpec((BR, DD), lambda i: (i, 0)),
        out_shape=jax.ShapeDtypeStruct((NN, DD), jnp.float32),
    )(parts, gprev, dinv, b, wnext)


def _tc3_body(p_ref, g_ref, dinv_ref, b_ref, batch_ref, wlin_ref, blin_ref,
              out_ref, acc_ref, cnt_ref):
    i = pl.program_id(0)

    @pl.when(i == 0)
    def _():
        acc_ref[...] = jnp.zeros_like(acc_ref)
        cnt_ref[...] = jnp.zeros_like(cnt_ref)

    p = p_ref[...]
    s = jnp.concatenate([p[0], p[1]], axis=1)
    h3 = (s + g_ref[...]) * dinv_ref[...] + b_ref[...]
    bidx = batch_ref[...]
    gids = lax.broadcasted_iota(jnp.int32, (1, GG), 1)
    mask = (bidx == gids).astype(jnp.float32)
    acc_ref[...] += lax.dot_general(
        mask, h3, (((0,), (0,)), ((), ())), preferred_element_type=jnp.float32
    )
    cnt_ref[...] += lax.dot_general(
        mask, jnp.ones((BR, 1), jnp.float32), (((0,), (0,)), ((), ())),
        preferred_element_type=jnp.float32,
    )

    @pl.when(i == pl.num_programs(0) - 1)
    def _():
        pooled = acc_ref[...] / jnp.maximum(cnt_ref[...], 1.0)
        out_ref[...] = (
            jnp.dot(pooled, wlin_ref[...], preferred_element_type=jnp.float32)
            + blin_ref[...]
        )


def _tc3(parts, g3, dinv, b3, batch2d, wlin, blin):
    return pl.pallas_call(
        _tc3_body,
        grid=(GRID,),
        in_specs=[
            pl.BlockSpec((2, BR, HD), lambda i: (0, i, 0)),
            pl.BlockSpec((BR, DD), lambda i: (i, 0)),
            pl.BlockSpec((BR, 1), lambda i: (i, 0)),
            pl.BlockSpec((1, DD), lambda i: (0, 0)),
            pl.BlockSpec((BR, 1), lambda i: (i, 0)),
            pl.BlockSpec((DD, 2), lambda i: (0, 0)),
            pl.BlockSpec((1, 2), lambda i: (0, 0)),
        ],
        out_specs=pl.BlockSpec((GG, 2), lambda i: (0, 0)),
        out_shape=jax.ShapeDtypeStruct((GG, 2), jnp.float32),
        scratch_shapes=[
            pltpu.VMEM((GG, DD), jnp.float32),
            pltpu.VMEM((GG, 1), jnp.float32),
        ],
    )(parts, g3, dinv, b3, batch2d, wlin, blin)


def kernel(x, edge_index, batch, W1, b1, W2, b2, W3, b3, Wlin, blin):
    src = edge_index[0].astype(jnp.int32)
    dst = edge_index[1].astype(jnp.int32)
    pad_e = EP - src.shape[0]
    srcf = jnp.concatenate([src, jnp.zeros((pad_e,), jnp.int32)])
    srcev = (2 * srcf).reshape(NW, EPW_CHUNKS, CHUNK)
    srcod = (2 * srcf + 1).reshape(NW, EPW_CHUNKS, CHUNK)
    dstp = jnp.concatenate([dst, jnp.full((pad_e,), NP - 1, jnp.int32)])
    dstp = dstp.reshape(NW, EPW_CHUNKS, CHUNK)
    ones16 = jnp.ones((CHUNK, 16), jnp.float32)
    zeros16 = jnp.zeros((CHUNK, 16), jnp.float32)
    zeros64 = jnp.zeros((CHUNK, HD), jnp.float32)
    batch2d = batch.astype(jnp.int32).reshape(NN, 1)

    degp = _run_deg(dstp, ones16, zeros16)
    g1, dinv = _tc0(x, W1, degp)
    parts1 = _run_prop(g1, srcev, srcod, dstp, zeros64)
    g2 = _tcmid(parts1, g1, dinv, b1.reshape(1, DD), W2)
    parts2 = _run_prop(g2, srcev, srcod, dstp, zeros64)
    g3 = _tcmid(parts2, g2, dinv, b2.reshape(1, DD), W3)
    parts3 = _run_prop(g3, srcev, srcod, dstp, zeros64)
    return _tc3(parts3, g3, dinv, b3.reshape(1, DD), batch2d, Wlin,
                blin.reshape(1, 2))


# speedup vs baseline: 9.2751x; 1.1442x over previous
import functools

import jax
import jax.numpy as jnp
from jax import lax
from jax.experimental import pallas as pl
from jax.experimental.pallas import tpu as pltpu
from jax.experimental.pallas import tpu_sc as plsc

NN = 10000
DD = 128
HD = 64
GG = 64
NP = 10240
NW = 32
TILES = 16
ROWS_PER_TILE = NP // TILES
CHUNK = 128
EPW_CHUNKS = 80
EP = NW * EPW_CHUNKS * CHUNK
BR = 1000
GRID = NN // BR

@functools.cache
def _sc_kernels():
    mesh = plsc.VectorSubcoreMesh(
        core_axis_name="c", subcore_axis_name="s", num_cores=2,
        num_subcores=TILES,
    )

    @functools.partial(
        pl.kernel,
        out_type=jax.ShapeDtypeStruct((2, NP, 16), jnp.float32),
        mesh=mesh,
        compiler_params=pltpu.CompilerParams(use_tc_tiling_on_sc=False),
        scratch_types=[
            pltpu.VMEM((EPW_CHUNKS, CHUNK), jnp.int32),
            pltpu.VMEM((CHUNK, 16), jnp.float32),
            pltpu.VMEM((CHUNK, 16), jnp.float32),
            pltpu.VMEM_SHARED((NP, 16), jnp.float32),
            pltpu.SemaphoreType.DMA,
        ],
    )
    def deg_kernel(dstp, ones16, zeros16, degp, dst_v, ones_v, zb, acc, sem):
        c = lax.axis_index("c")
        s = lax.axis_index("s")
        w = c * TILES + s
        pltpu.sync_copy(dstp.at[w], dst_v)
        pltpu.sync_copy(ones16, ones_v)
        pltpu.sync_copy(zeros16, zb)
        r0 = s * ROWS_PER_TILE
        for k in range(ROWS_PER_TILE // CHUNK):
            pltpu.sync_copy(zb, acc.at[pl.ds(r0 + k * CHUNK, CHUNK)])
        plsc.subcore_barrier()

        def body(j, carry):
            pltpu.sync_copy(ones_v, acc.at[dst_v.at[j]], add=True)
            return carry

        lax.fori_loop(0, EPW_CHUNKS, body, 0)
        plsc.subcore_barrier()
        for k in range(ROWS_PER_TILE // CHUNK):
            pltpu.sync_copy(acc.at[pl.ds(r0 + k * CHUNK, CHUNK)], zb)
            pltpu.sync_copy(zb, degp.at[c, pl.ds(r0 + k * CHUNK, CHUNK)])

    @functools.partial(
        pl.kernel,
        out_type=jax.ShapeDtypeStruct((2, NP, HD), jnp.float32),
        mesh=mesh,
        compiler_params=pltpu.CompilerParams(use_tc_tiling_on_sc=False),
        scratch_types=[
            pltpu.VMEM((2 * EPW_CHUNKS, CHUNK), jnp.int32),
            pltpu.VMEM((2 * EPW_CHUNKS, CHUNK), jnp.int32),
            pltpu.VMEM((CHUNK, HD), jnp.float32),
            pltpu.VMEM((CHUNK, HD), jnp.float32),
            pltpu.VMEM((CHUNK, HD), jnp.float32),
            pltpu.VMEM_SHARED((NP, HD), jnp.float32),
            pltpu.SemaphoreType.DMA,
            pltpu.SemaphoreType.DMA,
            pltpu.SemaphoreType.DMA,
            pltpu.SemaphoreType.DMA,
        ],
    )
    def prop_kernel(g2, srcev, srcod, dstp, zeros64, parts,
                    src_v, dst_v, rows0, rows1, zb, acc,
                    gsem0, gsem1, ssem0, ssem1):
        c = lax.axis_index("c")
        s = lax.axis_index("s")

        @pl.when(c == 0)
        def _():
            pltpu.sync_copy(srcev.at[2 * s], src_v.at[pl.ds(0, EPW_CHUNKS)])
            pltpu.sync_copy(srcev.at[2 * s + 1],
                            src_v.at[pl.ds(EPW_CHUNKS, EPW_CHUNKS)])

        @pl.when(c == 1)
        def _():
            pltpu.sync_copy(srcod.at[2 * s], src_v.at[pl.ds(0, EPW_CHUNKS)])
            pltpu.sync_copy(srcod.at[2 * s + 1],
                            src_v.at[pl.ds(EPW_CHUNKS, EPW_CHUNKS)])

        pltpu.sync_copy(dstp.at[2 * s], dst_v.at[pl.ds(0, EPW_CHUNKS)])
        pltpu.sync_copy(dstp.at[2 * s + 1],
                        dst_v.at[pl.ds(EPW_CHUNKS, EPW_CHUNKS)])
        pltpu.sync_copy(zeros64, zb)
        r0 = s * ROWS_PER_TILE
        for k in range(ROWS_PER_TILE // CHUNK):
            pltpu.sync_copy(zb, acc.at[pl.ds(r0 + k * CHUNK, CHUNK)])
        plsc.subcore_barrier()

        nch = 2 * EPW_CHUNKS
        pltpu.async_copy(g2.at[src_v.at[0]], rows0, gsem0)

        def body(i, carry):
            j = 2 * i
            pltpu.async_copy(g2.at[src_v.at[j + 1]], rows1, gsem1)
            pltpu.make_async_copy(g2.at[src_v.at[j]], rows0, gsem0).wait()
            pltpu.async_copy(rows0, acc.at[dst_v.at[j]], ssem0, add=True)

            @pl.when(i < nch // 2 - 1)
            def _():
                pltpu.make_async_copy(rows0, acc.at[dst_v.at[j]],
                                      ssem0).wait()
                pltpu.async_copy(g2.at[src_v.at[j + 2]], rows0, gsem0)

            pltpu.make_async_copy(g2.at[src_v.at[j + 1]], rows1,
                                  gsem1).wait()
            pltpu.async_copy(rows1, acc.at[dst_v.at[j + 1]], ssem1, add=True)
            pltpu.make_async_copy(rows1, acc.at[dst_v.at[j + 1]],
                                  ssem1).wait()
            return carry

        lax.fori_loop(0, nch // 2, body, 0)
        pltpu.make_async_copy(rows0, acc.at[dst_v.at[nch - 2]], ssem0).wait()
        plsc.subcore_barrier()
        for k in range(ROWS_PER_TILE // CHUNK):
            pltpu.sync_copy(acc.at[pl.ds(r0 + k * CHUNK, CHUNK)], zb)
            pltpu.sync_copy(zb, parts.at[c, pl.ds(r0 + k * CHUNK, CHUNK)])

    return deg_kernel, prop_kernel


def _run_deg(dstp, ones16, zeros16):
    return _sc_kernels()[0](dstp, ones16, zeros16)


def _run_prop(g, srcev, srcod, dstp, zeros64):
    g2 = g.reshape(2 * NN, HD)
    return _sc_kernels()[1](g2, srcev, srcod, dstp, zeros64)


def _tc0_body(x_ref, w_ref, degp_ref, g_ref, dinv_ref):
    dp = degp_ref[...]
    deg = dp[0, :, 0:1] + dp[1, :, 0:1] + 1.0
    dinv = 1.0 / jnp.sqrt(deg)
    h = jnp.dot(x_ref[...], w_ref[...], preferred_element_type=jnp.float32)
    g_ref[...] = h * dinv
    dinv_ref[...] = dinv


def _tc0(x, w1, degp):
    return pl.pallas_call(
        _tc0_body,
        grid=(GRID,),
        in_specs=[
            pl.BlockSpec((BR, DD), lambda i: (i, 0)),
            pl.BlockSpec((DD, DD), lambda i: (0, 0)),
            pl.BlockSpec((2, BR, 16), lambda i: (0, i, 0)),
        ],
        out_specs=[
            pl.BlockSpec((BR, DD), lambda i: (i, 0)),
            pl.BlockSpec((BR, 1), lambda i: (i, 0)),
        ],
        out_shape=[
            jax.ShapeDtypeStruct((NN, DD), jnp.float32),
            jax.ShapeDtypeStruct((NN, 1), jnp.float32),
        ],
    )(x, w1, degp)


def _tcmid_body(p_ref, g_ref, dinv_ref, b_ref, w_ref, out_ref):
    p = p_ref[...]
    s = jnp.concatenate([p[0], p[1]], axis=1)
    dinv = dinv_ref[...]
    xb = (s + g_ref[...]) * dinv + b_ref[...]
    xb = jnp.maximum(xb, 0.0)
    out_ref[...] = (
        jnp.dot(xb, w_ref[...], preferred_element_type=jnp.float32) * dinv
    )


def _tcmid(parts, gprev, dinv, b, wnext):
    return pl.pallas_call(
        _tcmid_body,
        grid=(GRID,),
        in_specs=[
            pl.BlockSpec((2, BR, HD), lambda i: (0, i, 0)),
            pl.BlockSpec((BR, DD), lambda i: (i, 0)),
            pl.BlockSpec((BR, 1), lambda i: (i, 0)),
            pl.BlockSpec((1, DD), lambda i: (0, 0)),
            pl.BlockSpec((DD, DD), lambda i: (0, 0)),
        ],
        out_specs=pl.BlockSpec((BR, DD), lambda i: (i, 0)),
        out_shape=jax.ShapeDtypeStruct((NN, DD), jnp.float32),
    )(parts, gprev, dinv, b, wnext)


def _tc3_body(p_ref, g_ref, dinv_ref, b_ref, batch_ref, wlin_ref, blin_ref,
              out_ref, acc_ref, cnt_ref):
    i = pl.program_id(0)

    @pl.when(i == 0)
    def _():
        acc_ref[...] = jnp.zeros_like(acc_ref)
        cnt_ref[...] = jnp.zeros_like(cnt_ref)

    p = p_ref[...]
    s = jnp.concatenate([p[0], p[1]], axis=1)
    h3 = (s + g_ref[...]) * dinv_ref[...] + b_ref[...]
    bidx = batch_ref[...]
    gids = lax.broadcasted_iota(jnp.int32, (1, GG), 1)
    mask = (bidx == gids).astype(jnp.float32)
    acc_ref[...] += lax.dot_general(
        mask, h3, (((0,), (0,)), ((), ())), preferred_element_type=jnp.float32
    )
    cnt_ref[...] += lax.dot_general(
        mask, jnp.ones((BR, 1), jnp.float32), (((0,), (0,)), ((), ())),
        preferred_element_type=jnp.float32,
    )

    @pl.when(i == pl.num_programs(0) - 1)
    def _():
        pooled = acc_ref[...] / jnp.maximum(cnt_ref[...], 1.0)
        out_ref[...] = (
            jnp.dot(pooled, wlin_ref[...], preferred_element_type=jnp.float32)
            + blin_ref[...]
        )


def _tc3(parts, g3, dinv, b3, batch2d, wlin, blin):
    return pl.pallas_call(
        _tc3_body,
        grid=(GRID,),
        in_specs=[
            pl.BlockSpec((2, BR, HD), lambda i: (0, i, 0)),
            pl.BlockSpec((BR, DD), lambda i: (i, 0)),
            pl.BlockSpec((BR, 1), lambda i: (i, 0)),
            pl.BlockSpec((1, DD), lambda i: (0, 0)),
            pl.BlockSpec((BR, 1), lambda i: (i, 0)),
            pl.BlockSpec((DD, 2), lambda i: (0, 0)),
            pl.BlockSpec((1, 2), lambda i: (0, 0)),
        ],
        out_specs=pl.BlockSpec((GG, 2), lambda i: (0, 0)),
        out_shape=jax.ShapeDtypeStruct((GG, 2), jnp.float32),
        scratch_shapes=[
            pltpu.VMEM((GG, DD), jnp.float32),
            pltpu.VMEM((GG, 1), jnp.float32),
        ],
    )(parts, g3, dinv, b3, batch2d, wlin, blin)


def kernel(x, edge_index, batch, W1, b1, W2, b2, W3, b3, Wlin, blin):
    src = edge_index[0].astype(jnp.int32)
    dst = edge_index[1].astype(jnp.int32)
    pad_e = EP - src.shape[0]
    srcf = jnp.concatenate([src, jnp.zeros((pad_e,), jnp.int32)])
    srcev = (2 * srcf).reshape(NW, EPW_CHUNKS, CHUNK)
    srcod = (2 * srcf + 1).reshape(NW, EPW_CHUNKS, CHUNK)
    dstp = jnp.concatenate([dst, jnp.full((pad_e,), NP - 1, jnp.int32)])
    dstp = dstp.reshape(NW, EPW_CHUNKS, CHUNK)
    ones16 = jnp.ones((CHUNK, 16), jnp.float32)
    zeros16 = jnp.zeros((CHUNK, 16), jnp.float32)
    zeros64 = jnp.zeros((CHUNK, HD), jnp.float32)
    batch2d = batch.astype(jnp.int32).reshape(NN, 1)

    degp = _run_deg(dstp, ones16, zeros16)
    g1, dinv = _tc0(x, W1, degp)
    parts1 = _run_prop(g1, srcev, srcod, dstp, zeros64)
    g2 = _tcmid(parts1, g1, dinv, b1.reshape(1, DD), W2)
    parts2 = _run_prop(g2, srcev, srcod, dstp, zeros64)
    g3 = _tcmid(parts2, g2, dinv, b2.reshape(1, DD), W3)
    parts3 = _run_prop(g3, srcev, srcod, dstp, zeros64)
    return _tc3(parts3, g3, dinv, b3.reshape(1, DD), batch2d, Wlin,
                blin.reshape(1, 2))


# speedup vs baseline: 9.7274x; 1.0488x over previous
import functools

import jax
import jax.numpy as jnp
from jax import lax
from jax.experimental import pallas as pl
from jax.experimental.pallas import tpu as pltpu
from jax.experimental.pallas import tpu_sc as plsc

NN = 10000
DD = 128
HD = 64
GG = 64
NP = 10240
NW = 32
TILES = 16
ROWS_PER_TILE = NP // TILES
CHUNK = 128
EPW_CHUNKS = 80
EP = NW * EPW_CHUNKS * CHUNK
BR = 1000
GRID = NN // BR

@functools.cache
def _sc_kernels():
    mesh = plsc.VectorSubcoreMesh(
        core_axis_name="c", subcore_axis_name="s", num_cores=2,
        num_subcores=TILES,
    )

    @functools.partial(
        pl.kernel,
        out_type=jax.ShapeDtypeStruct((2, NP, 16), jnp.float32),
        mesh=mesh,
        compiler_params=pltpu.CompilerParams(use_tc_tiling_on_sc=False),
        scratch_types=[
            pltpu.VMEM((EPW_CHUNKS, CHUNK), jnp.int32),
            pltpu.VMEM((CHUNK, 16), jnp.float32),
            pltpu.VMEM((CHUNK, 16), jnp.float32),
            pltpu.VMEM_SHARED((NP, 16), jnp.float32),
            pltpu.SemaphoreType.DMA,
        ],
    )
    def deg_kernel(dstp, ones16, zeros16, degp, dst_v, ones_v, zb, acc, sem):
        c = lax.axis_index("c")
        s = lax.axis_index("s")
        w = c * TILES + s
        pltpu.sync_copy(dstp.at[w], dst_v)
        pltpu.sync_copy(ones16, ones_v)
        pltpu.sync_copy(zeros16, zb)
        r0 = s * ROWS_PER_TILE
        for k in range(ROWS_PER_TILE // CHUNK):
            pltpu.sync_copy(zb, acc.at[pl.ds(r0 + k * CHUNK, CHUNK)])
        plsc.subcore_barrier()

        def body(j, carry):
            pltpu.sync_copy(ones_v, acc.at[dst_v.at[j]], add=True)
            return carry

        lax.fori_loop(0, EPW_CHUNKS, body, 0)
        plsc.subcore_barrier()
        for k in range(ROWS_PER_TILE // CHUNK):
            pltpu.sync_copy(acc.at[pl.ds(r0 + k * CHUNK, CHUNK)], zb)
            pltpu.sync_copy(zb, degp.at[c, pl.ds(r0 + k * CHUNK, CHUNK)])

    @functools.partial(
        pl.kernel,
        out_type=jax.ShapeDtypeStruct((2, NP, HD), jnp.float32),
        mesh=mesh,
        compiler_params=pltpu.CompilerParams(use_tc_tiling_on_sc=False),
        scratch_types=[
            pltpu.VMEM((2 * EPW_CHUNKS, CHUNK), jnp.int32),
            pltpu.VMEM((2 * EPW_CHUNKS, CHUNK), jnp.int32),
            pltpu.VMEM((4, CHUNK, HD), jnp.float32),
            pltpu.VMEM((CHUNK, HD), jnp.float32),
            pltpu.VMEM_SHARED((NP, HD), jnp.float32),
            pltpu.SemaphoreType.DMA,
            pltpu.SemaphoreType.DMA,
            pltpu.SemaphoreType.DMA,
            pltpu.SemaphoreType.DMA,
            pltpu.SemaphoreType.DMA,
            pltpu.SemaphoreType.DMA,
            pltpu.SemaphoreType.DMA,
            pltpu.SemaphoreType.DMA,
        ],
    )
    def prop_kernel(g2, srcev, srcod, dstp, zeros64, parts,
                    src_v, dst_v, rowsr, zb, acc,
                    gsem0, gsem1, gsem2, gsem3,
                    ssem0, ssem1, ssem2, ssem3):
        gsems = (gsem0, gsem1, gsem2, gsem3)
        ssems = (ssem0, ssem1, ssem2, ssem3)
        c = lax.axis_index("c")
        s = lax.axis_index("s")

        @pl.when(c == 0)
        def _():
            pltpu.sync_copy(srcev.at[2 * s], src_v.at[pl.ds(0, EPW_CHUNKS)])
            pltpu.sync_copy(srcev.at[2 * s + 1],
                            src_v.at[pl.ds(EPW_CHUNKS, EPW_CHUNKS)])

        @pl.when(c == 1)
        def _():
            pltpu.sync_copy(srcod.at[2 * s], src_v.at[pl.ds(0, EPW_CHUNKS)])
            pltpu.sync_copy(srcod.at[2 * s + 1],
                            src_v.at[pl.ds(EPW_CHUNKS, EPW_CHUNKS)])

        pltpu.sync_copy(dstp.at[2 * s], dst_v.at[pl.ds(0, EPW_CHUNKS)])
        pltpu.sync_copy(dstp.at[2 * s + 1],
                        dst_v.at[pl.ds(EPW_CHUNKS, EPW_CHUNKS)])
        pltpu.sync_copy(zeros64, zb)
        r0 = s * ROWS_PER_TILE
        for k in range(ROWS_PER_TILE // CHUNK):
            pltpu.sync_copy(zb, acc.at[pl.ds(r0 + k * CHUNK, CHUNK)])
        plsc.subcore_barrier()

        nch = 2 * EPW_CHUNKS
        NB = 4
        for b in range(NB - 1):
            pltpu.async_copy(g2.at[src_v.at[b]], rowsr.at[b], gsems[b])

        def body(i, carry):
            for b in range(NB):
                jj = NB * i + b
                nxt = (b + NB - 1) % NB

                @pl.when((jj + NB - 1 < nch) & (jj > 0))
                def _():
                    pltpu.make_async_copy(
                        rowsr.at[nxt], acc.at[dst_v.at[jj - 1]],
                        ssems[nxt]).wait()

                @pl.when(jj + NB - 1 < nch)
                def _():
                    pltpu.async_copy(g2.at[src_v.at[jj + NB - 1]],
                                     rowsr.at[nxt], gsems[nxt])

                pltpu.make_async_copy(g2.at[src_v.at[jj]], rowsr.at[b],
                                      gsems[b]).wait()
                pltpu.async_copy(rowsr.at[b], acc.at[dst_v.at[jj]],
                                 ssems[b], add=True)
            return carry

        lax.fori_loop(0, nch // NB, body, 0)
        for b in range(NB):
            pltpu.make_async_copy(rowsr.at[b],
                                  acc.at[dst_v.at[nch - NB + b]],
                                  ssems[b]).wait()
        plsc.subcore_barrier()
        for k in range(ROWS_PER_TILE // CHUNK):
            pltpu.sync_copy(acc.at[pl.ds(r0 + k * CHUNK, CHUNK)], zb)
            pltpu.sync_copy(zb, parts.at[c, pl.ds(r0 + k * CHUNK, CHUNK)])

    return deg_kernel, prop_kernel


def _run_deg(dstp, ones16, zeros16):
    return _sc_kernels()[0](dstp, ones16, zeros16)


def _run_prop(g, srcev, srcod, dstp, zeros64):
    g2 = g.reshape(2 * NN, HD)
    return _sc_kernels()[1](g2, srcev, srcod, dstp, zeros64)


def _tc0_body(x_ref, w_ref, degp_ref, g_ref, dinv_ref):
    dp = degp_ref[...]
    deg = dp[0, :, 0:1] + dp[1, :, 0:1] + 1.0
    dinv = 1.0 / jnp.sqrt(deg)
    h = jnp.dot(x_ref[...], w_ref[...], preferred_element_type=jnp.float32)
    g_ref[...] = h * dinv
    dinv_ref[...] = dinv


def _tc0(x, w1, degp):
    return pl.pallas_call(
        _tc0_body,
        grid=(GRID,),
        in_specs=[
            pl.BlockSpec((BR, DD), lambda i: (i, 0)),
            pl.BlockSpec((DD, DD), lambda i: (0, 0)),
            pl.BlockSpec((2, BR, 16), lambda i: (0, i, 0)),
        ],
        out_specs=[
            pl.BlockSpec((BR, DD), lambda i: (i, 0)),
            pl.BlockSpec((BR, 1), lambda i: (i, 0)),
        ],
        out_shape=[
            jax.ShapeDtypeStruct((NN, DD), jnp.float32),
            jax.ShapeDtypeStruct((NN, 1), jnp.float32),
        ],
    )(x, w1, degp)


def _tcmid_body(p_ref, g_ref, dinv_ref, b_ref, w_ref, out_ref):
    p = p_ref[...]
    s = jnp.concatenate([p[0], p[1]], axis=1)
    dinv = dinv_ref[...]
    xb = (s + g_ref[...]) * dinv + b_ref[...]
    xb = jnp.maximum(xb, 0.0)
    out_ref[...] = (
        jnp.dot(xb, w_ref[...], preferred_element_type=jnp.float32) * dinv
    )


def _tcmid(parts, gprev, dinv, b, wnext):
    return pl.pallas_call(
        _tcmid_body,
        grid=(GRID,),
        in_specs=[
            pl.BlockSpec((2, BR, HD), lambda i: (0, i, 0)),
            pl.BlockSpec((BR, DD), lambda i: (i, 0)),
            pl.BlockSpec((BR, 1), lambda i: (i, 0)),
            pl.BlockSpec((1, DD), lambda i: (0, 0)),
            pl.BlockSpec((DD, DD), lambda i: (0, 0)),
        ],
        out_specs=pl.BlockSpec((BR, DD), lambda i: (i, 0)),
        out_shape=jax.ShapeDtypeStruct((NN, DD), jnp.float32),
    )(parts, gprev, dinv, b, wnext)


def _tc3_body(p_ref, g_ref, dinv_ref, b_ref, batch_ref, wlin_ref, blin_ref,
              out_ref, acc_ref, cnt_ref):
    i = pl.program_id(0)

    @pl.when(i == 0)
    def _():
        acc_ref[...] = jnp.zeros_like(acc_ref)
        cnt_ref[...] = jnp.zeros_like(cnt_ref)

    p = p_ref[...]
    s = jnp.concatenate([p[0], p[1]], axis=1)
    h3 = (s + g_ref[...]) * dinv_ref[...] + b_ref[...]
    bidx = batch_ref[...]
    gids = lax.broadcasted_iota(jnp.int32, (1, GG), 1)
    mask = (bidx == gids).astype(jnp.float32)
    acc_ref[...] += lax.dot_general(
        mask, h3, (((0,), (0,)), ((), ())), preferred_element_type=jnp.float32
    )
    cnt_ref[...] += lax.dot_general(
        mask, jnp.ones((BR, 1), jnp.float32), (((0,), (0,)), ((), ())),
        preferred_element_type=jnp.float32,
    )

    @pl.when(i == pl.num_programs(0) - 1)
    def _():
        pooled = acc_ref[...] / jnp.maximum(cnt_ref[...], 1.0)
        out_ref[...] = (
            jnp.dot(pooled, wlin_ref[...], preferred_element_type=jnp.float32)
            + blin_ref[...]
        )


def _tc3(parts, g3, dinv, b3, batch2d, wlin, blin):
    return pl.pallas_call(
        _tc3_body,
        grid=(GRID,),
        in_specs=[
            pl.BlockSpec((2, BR, HD), lambda i: (0, i, 0)),
            pl.BlockSpec((BR, DD), lambda i: (i, 0)),
            pl.BlockSpec((BR, 1), lambda i: (i, 0)),
            pl.BlockSpec((1, DD), lambda i: (0, 0)),
            pl.BlockSpec((BR, 1), lambda i: (i, 0)),
            pl.BlockSpec((DD, 2), lambda i: (0, 0)),
            pl.BlockSpec((1, 2), lambda i: (0, 0)),
        ],
        out_specs=pl.BlockSpec((GG, 2), lambda i: (0, 0)),
        out_shape=jax.ShapeDtypeStruct((GG, 2), jnp.float32),
        scratch_shapes=[
            pltpu.VMEM((GG, DD), jnp.float32),
            pltpu.VMEM((GG, 1), jnp.float32),
        ],
    )(parts, g3, dinv, b3, batch2d, wlin, blin)


def kernel(x, edge_index, batch, W1, b1, W2, b2, W3, b3, Wlin, blin):
    src = edge_index[0].astype(jnp.int32)
    dst = edge_index[1].astype(jnp.int32)
    pad_e = EP - src.shape[0]
    srcf = jnp.concatenate([src, jnp.zeros((pad_e,), jnp.int32)])
    srcev = (2 * srcf).reshape(NW, EPW_CHUNKS, CHUNK)
    srcod = (2 * srcf + 1).reshape(NW, EPW_CHUNKS, CHUNK)
    dstp = jnp.concatenate([dst, jnp.full((pad_e,), NP - 1, jnp.int32)])
    dstp = dstp.reshape(NW, EPW_CHUNKS, CHUNK)
    ones16 = jnp.ones((CHUNK, 16), jnp.float32)
    zeros16 = jnp.zeros((CHUNK, 16), jnp.float32)
    zeros64 = jnp.zeros((CHUNK, HD), jnp.float32)
    batch2d = batch.astype(jnp.int32).reshape(NN, 1)

    degp = _run_deg(dstp, ones16, zeros16)
    g1, dinv = _tc0(x, W1, degp)
    parts1 = _run_prop(g1, srcev, srcod, dstp, zeros64)
    g2 = _tcmid(parts1, g1, dinv, b1.reshape(1, DD), W2)
    parts2 = _run_prop(g2, srcev, srcod, dstp, zeros64)
    g3 = _tcmid(parts2, g2, dinv, b2.reshape(1, DD), W3)
    parts3 = _run_prop(g3, srcev, srcod, dstp, zeros64)
    return _tc3(parts3, g3, dinv, b3.reshape(1, DD), batch2d, Wlin,
                blin.reshape(1, 2))


# speedup vs baseline: 18.2977x; 1.8810x over previous
import functools

import jax
import jax.numpy as jnp
from jax import lax
from jax.experimental import pallas as pl
from jax.experimental.pallas import tpu as pltpu
from jax.experimental.pallas import tpu_sc as plsc

NN = 10000
DD = 128
HD = 64
QD = 32
GG = 64
NP = 10240
NW = 32
TILES = 16
ROWS_PER_TILE = NP // TILES
CHUNK = 128
EPW_CHUNKS = 80
EP = NW * EPW_CHUNKS * CHUNK
BR = 1000
GRID = NN // BR

@functools.cache
def _sc_kernels():
    mesh = plsc.VectorSubcoreMesh(
        core_axis_name="c", subcore_axis_name="s", num_cores=2,
        num_subcores=TILES,
    )

    @functools.partial(
        pl.kernel,
        out_type=jax.ShapeDtypeStruct((2, NP, 16), jnp.float32),
        mesh=mesh,
        compiler_params=pltpu.CompilerParams(use_tc_tiling_on_sc=False),
        scratch_types=[
            pltpu.VMEM((EPW_CHUNKS, CHUNK), jnp.int32),
            pltpu.VMEM((CHUNK, 16), jnp.float32),
            pltpu.VMEM((CHUNK, 16), jnp.float32),
            pltpu.VMEM_SHARED((NP, 16), jnp.float32),
            pltpu.SemaphoreType.DMA,
        ],
    )
    def deg_kernel(dstp, ones16, zeros16, degp, dst_v, ones_v, zb, acc, sem):
        c = lax.axis_index("c")
        s = lax.axis_index("s")
        w = c * TILES + s
        pltpu.sync_copy(dstp.at[w], dst_v)
        pltpu.sync_copy(ones16, ones_v)
        pltpu.sync_copy(zeros16, zb)
        r0 = s * ROWS_PER_TILE
        for k in range(ROWS_PER_TILE // CHUNK):
            pltpu.sync_copy(zb, acc.at[pl.ds(r0 + k * CHUNK, CHUNK)])
        plsc.subcore_barrier()

        def body(j, carry):
            pltpu.sync_copy(ones_v, acc.at[dst_v.at[j]], add=True)
            return carry

        lax.fori_loop(0, EPW_CHUNKS, body, 0)
        plsc.subcore_barrier()
        for k in range(ROWS_PER_TILE // CHUNK):
            pltpu.sync_copy(acc.at[pl.ds(r0 + k * CHUNK, CHUNK)], zb)
            pltpu.sync_copy(zb, degp.at[c, pl.ds(r0 + k * CHUNK, CHUNK)])

    @functools.partial(
        pl.kernel,
        out_type=jax.ShapeDtypeStruct((2, 2, NP, QD), jnp.float32),
        mesh=mesh,
        compiler_params=pltpu.CompilerParams(use_tc_tiling_on_sc=False),
        scratch_types=[
            pltpu.VMEM((2 * EPW_CHUNKS, CHUNK), jnp.int32),
            pltpu.VMEM((2 * EPW_CHUNKS, CHUNK), jnp.int32),
            pltpu.VMEM((4, CHUNK, QD), jnp.float32),
            pltpu.VMEM((CHUNK, QD), jnp.float32),
            pltpu.VMEM((CHUNK, QD), jnp.float32),
            pltpu.VMEM_SHARED((NP, QD), jnp.float32),
            pltpu.VMEM_SHARED((NP, QD), jnp.float32),
            pltpu.VMEM_SHARED((NP, QD), jnp.float32),
            pltpu.SemaphoreType.DMA,
            pltpu.SemaphoreType.DMA,
            pltpu.SemaphoreType.DMA,
            pltpu.SemaphoreType.DMA,
            pltpu.SemaphoreType.DMA,
            pltpu.SemaphoreType.DMA,
            pltpu.SemaphoreType.DMA,
            pltpu.SemaphoreType.DMA,
        ],
    )
    def prop_kernel(gq, srcp, dstp, zeros32, parts,
                    src_v, dst_v, rowsr, zbz, zbs, tabA, tabB, acc,
                    gsem0, gsem1, gsem2, gsem3,
                    ssem0, ssem1, ssem2, ssem3):
        gsems = (gsem0, gsem1, gsem2, gsem3)
        ssems = (ssem0, ssem1, ssem2, ssem3)
        c = lax.axis_index("c")
        s = lax.axis_index("s")

        pltpu.sync_copy(srcp.at[2 * s], src_v.at[pl.ds(0, EPW_CHUNKS)])
        pltpu.sync_copy(srcp.at[2 * s + 1],
                        src_v.at[pl.ds(EPW_CHUNKS, EPW_CHUNKS)])
        pltpu.sync_copy(dstp.at[2 * s], dst_v.at[pl.ds(0, EPW_CHUNKS)])
        pltpu.sync_copy(dstp.at[2 * s + 1],
                        dst_v.at[pl.ds(EPW_CHUNKS, EPW_CHUNKS)])
        pltpu.sync_copy(zeros32, zbz)
        r0 = s * ROWS_PER_TILE
        pltpu.sync_copy(gq.at[2 * c, pl.ds(r0, ROWS_PER_TILE)],
                        tabA.at[pl.ds(r0, ROWS_PER_TILE)])
        pltpu.sync_copy(gq.at[2 * c + 1, pl.ds(r0, ROWS_PER_TILE)],
                        tabB.at[pl.ds(r0, ROWS_PER_TILE)])
        for k in range(ROWS_PER_TILE // CHUNK):
            pltpu.sync_copy(zbz, acc.at[pl.ds(r0 + k * CHUNK, CHUNK)])
        plsc.subcore_barrier()

        nch = 2 * EPW_CHUNKS
        NB = 4
        for p, tab in enumerate((tabA, tabB)):
            for b in range(NB - 1):
                pltpu.async_copy(tab.at[src_v.at[b]], rowsr.at[b],
                                 gsems[b])

            def body(i, carry):
                for b in range(NB):
                    jj = NB * i + b
                    nxt = (b + NB - 1) % NB

                    @pl.when((jj + NB - 1 < nch) & (jj > 0))
                    def _():
                        pltpu.make_async_copy(
                            rowsr.at[nxt], acc.at[dst_v.at[jj - 1]],
                            ssems[nxt]).wait()

                    @pl.when(jj + NB - 1 < nch)
                    def _():
                        pltpu.async_copy(tab.at[src_v.at[jj + NB - 1]],
                                         rowsr.at[nxt], gsems[nxt])

                    pltpu.make_async_copy(tab.at[src_v.at[jj]],
                                          rowsr.at[b], gsems[b]).wait()
                    pltpu.async_copy(rowsr.at[b], acc.at[dst_v.at[jj]],
                                     ssems[b], add=True)
                return carry

            lax.fori_loop(0, nch // NB, body, 0)
            for b in range(NB):
                pltpu.make_async_copy(rowsr.at[b],
                                      acc.at[dst_v.at[nch - NB + b]],
                                      ssems[b]).wait()
            plsc.subcore_barrier()
            for k in range(ROWS_PER_TILE // CHUNK):
                pltpu.sync_copy(acc.at[pl.ds(r0 + k * CHUNK, CHUNK)], zbs)
                pltpu.sync_copy(zbs,
                                parts.at[c, p, pl.ds(r0 + k * CHUNK, CHUNK)])
                if p == 0:
                    pltpu.sync_copy(zbz,
                                    acc.at[pl.ds(r0 + k * CHUNK, CHUNK)])
            if p == 0:
                plsc.subcore_barrier()

    return deg_kernel, prop_kernel


def _run_deg(dstp, ones16, zeros16):
    return _sc_kernels()[0](dstp, ones16, zeros16)


def _run_prop(gq, srcp, dstp, zeros32):
    return _sc_kernels()[1](gq, srcp, dstp, zeros32)


def _tc0_body(x_ref, w_ref, degp_ref, g_ref, gq_ref, dinv_ref):
    dp = degp_ref[...]
    deg = dp[0, :, 0:1] + dp[1, :, 0:1] + 1.0
    dinv = 1.0 / jnp.sqrt(deg)
    h = jnp.dot(x_ref[...], w_ref[...], preferred_element_type=jnp.float32)
    g = h * dinv
    g_ref[...] = g
    gq_ref[...] = jnp.transpose(g.reshape(BR, 4, QD), (1, 0, 2))
    dinv_ref[...] = dinv


def _tc0(x, w1, degp):
    return pl.pallas_call(
        _tc0_body,
        grid=(GRID,),
        in_specs=[
            pl.BlockSpec((BR, DD), lambda i: (i, 0)),
            pl.BlockSpec((DD, DD), lambda i: (0, 0)),
            pl.BlockSpec((2, BR, 16), lambda i: (0, i, 0)),
        ],
        out_specs=[
            pl.BlockSpec((BR, DD), lambda i: (i, 0)),
            pl.BlockSpec((4, BR, QD), lambda i: (0, i, 0)),
            pl.BlockSpec((BR, 1), lambda i: (i, 0)),
        ],
        out_shape=[
            jax.ShapeDtypeStruct((NN, DD), jnp.float32),
            jax.ShapeDtypeStruct((4, NP, QD), jnp.float32),
            jax.ShapeDtypeStruct((NN, 1), jnp.float32),
        ],
    )(x, w1, degp)


def _pcat(p):
    return jnp.concatenate([p[0, 0], p[0, 1], p[1, 0], p[1, 1]], axis=1)


def _tcmid_body(p_ref, g_ref, dinv_ref, b_ref, w_ref, out_ref, gq_ref):
    s = _pcat(p_ref[...])
    dinv = dinv_ref[...]
    xb = (s + g_ref[...]) * dinv + b_ref[...]
    xb = jnp.maximum(xb, 0.0)
    g = jnp.dot(xb, w_ref[...], preferred_element_type=jnp.float32) * dinv
    out_ref[...] = g
    gq_ref[...] = jnp.transpose(g.reshape(BR, 4, QD), (1, 0, 2))


def _tcmid(parts, gprev, dinv, b, wnext):
    return pl.pallas_call(
        _tcmid_body,
        grid=(GRID,),
        in_specs=[
            pl.BlockSpec((2, 2, BR, QD), lambda i: (0, 0, i, 0)),
            pl.BlockSpec((BR, DD), lambda i: (i, 0)),
            pl.BlockSpec((BR, 1), lambda i: (i, 0)),
            pl.BlockSpec((1, DD), lambda i: (0, 0)),
            pl.BlockSpec((DD, DD), lambda i: (0, 0)),
        ],
        out_specs=[
            pl.BlockSpec((BR, DD), lambda i: (i, 0)),
            pl.BlockSpec((4, BR, QD), lambda i: (0, i, 0)),
        ],
        out_shape=[
            jax.ShapeDtypeStruct((NN, DD), jnp.float32),
            jax.ShapeDtypeStruct((4, NP, QD), jnp.float32),
        ],
    )(parts, gprev, dinv, b, wnext)


def _tc3_body(p_ref, g_ref, dinv_ref, b_ref, batch_ref, wlin_ref, blin_ref,
              out_ref, acc_ref, cnt_ref):
    i = pl.program_id(0)

    @pl.when(i == 0)
    def _():
        acc_ref[...] = jnp.zeros_like(acc_ref)
        cnt_ref[...] = jnp.zeros_like(cnt_ref)

    s = _pcat(p_ref[...])
    h3 = (s + g_ref[...]) * dinv_ref[...] + b_ref[...]
    bidx = batch_ref[...]
    gids = lax.broadcasted_iota(jnp.int32, (1, GG), 1)
    mask = (bidx == gids).astype(jnp.float32)
    acc_ref[...] += lax.dot_general(
        mask, h3, (((0,), (0,)), ((), ())), preferred_element_type=jnp.float32
    )
    cnt_ref[...] += lax.dot_general(
        mask, jnp.ones((BR, 1), jnp.float32), (((0,), (0,)), ((), ())),
        preferred_element_type=jnp.float32,
    )

    @pl.when(i == pl.num_programs(0) - 1)
    def _():
        pooled = acc_ref[...] / jnp.maximum(cnt_ref[...], 1.0)
        out_ref[...] = (
            jnp.dot(pooled, wlin_ref[...], preferred_element_type=jnp.float32)
            + blin_ref[...]
        )


def _tc3(parts, g3, dinv, b3, batch2d, wlin, blin):
    return pl.pallas_call(
        _tc3_body,
        grid=(GRID,),
        in_specs=[
            pl.BlockSpec((2, 2, BR, QD), lambda i: (0, 0, i, 0)),
            pl.BlockSpec((BR, DD), lambda i: (i, 0)),
            pl.BlockSpec((BR, 1), lambda i: (i, 0)),
            pl.BlockSpec((1, DD), lambda i: (0, 0)),
            pl.BlockSpec((BR, 1), lambda i: (i, 0)),
            pl.BlockSpec((DD, 2), lambda i: (0, 0)),
            pl.BlockSpec((1, 2), lambda i: (0, 0)),
        ],
        out_specs=pl.BlockSpec((GG, 2), lambda i: (0, 0)),
        out_shape=jax.ShapeDtypeStruct((GG, 2), jnp.float32),
        scratch_shapes=[
            pltpu.VMEM((GG, DD), jnp.float32),
            pltpu.VMEM((GG, 1), jnp.float32),
        ],
    )(parts, g3, dinv, b3, batch2d, wlin, blin)


def kernel(x, edge_index, batch, W1, b1, W2, b2, W3, b3, Wlin, blin):
    src = edge_index[0].astype(jnp.int32)
    dst = edge_index[1].astype(jnp.int32)
    pad_e = EP - src.shape[0]
    srcp = jnp.concatenate([src, jnp.zeros((pad_e,), jnp.int32)])
    srcp = srcp.reshape(NW, EPW_CHUNKS, CHUNK)
    dstp = jnp.concatenate([dst, jnp.full((pad_e,), NP - 1, jnp.int32)])
    dstp = dstp.reshape(NW, EPW_CHUNKS, CHUNK)
    ones16 = jnp.ones((CHUNK, 16), jnp.float32)
    zeros16 = jnp.zeros((CHUNK, 16), jnp.float32)
    zeros32 = jnp.zeros((CHUNK, QD), jnp.float32)
    batch2d = batch.astype(jnp.int32).reshape(NN, 1)

    degp = _run_deg(dstp, ones16, zeros16)
    g1, gq1, dinv = _tc0(x, W1, degp)
    parts1 = _run_prop(gq1, srcp, dstp, zeros32)
    g2, gq2 = _tcmid(parts1, g1, dinv, b1.reshape(1, DD), W2)
    parts2 = _run_prop(gq2, srcp, dstp, zeros32)
    g3, gq3 = _tcmid(parts2, g2, dinv, b2.reshape(1, DD), W3)
    parts3 = _run_prop(gq3, srcp, dstp, zeros32)
    return _tc3(parts3, g3, dinv, b3.reshape(1, DD), batch2d, Wlin,
                blin.reshape(1, 2))


# speedup vs baseline: 19.9823x; 1.0921x over previous
import functools

import jax
import jax.numpy as jnp
from jax import lax
from jax.experimental import pallas as pl
from jax.experimental.pallas import tpu as pltpu
from jax.experimental.pallas import tpu_sc as plsc

NN = 10000
DD = 128
HD = 64
QD = 32
GG = 64
NP = 10240
NW = 32
TILES = 16
ROWS_PER_TILE = NP // TILES
CHUNK = 128
EPW_CHUNKS = 80
EP = NW * EPW_CHUNKS * CHUNK
BR = 1000
GRID = NN // BR

@functools.cache
def _sc_kernels():
    mesh = plsc.VectorSubcoreMesh(
        core_axis_name="c", subcore_axis_name="s", num_cores=2,
        num_subcores=TILES,
    )

    @functools.partial(
        pl.kernel,
        out_type=jax.ShapeDtypeStruct((2, NP, 16), jnp.float32),
        mesh=mesh,
        compiler_params=pltpu.CompilerParams(use_tc_tiling_on_sc=False),
        scratch_types=[
            pltpu.VMEM((EPW_CHUNKS, CHUNK), jnp.int32),
            pltpu.VMEM((CHUNK, 16), jnp.float32),
            pltpu.VMEM((CHUNK, 16), jnp.float32),
            pltpu.VMEM_SHARED((NP, 16), jnp.float32),
            pltpu.SemaphoreType.DMA,
        ],
    )
    def deg_kernel(dstp, ones16, zeros16, degp, dst_v, ones_v, zb, acc, sem):
        c = lax.axis_index("c")
        s = lax.axis_index("s")
        w = c * TILES + s
        pltpu.sync_copy(dstp.at[w], dst_v)
        pltpu.sync_copy(ones16, ones_v)
        pltpu.sync_copy(zeros16, zb)
        r0 = s * ROWS_PER_TILE
        for k in range(ROWS_PER_TILE // CHUNK):
            pltpu.sync_copy(zb, acc.at[pl.ds(r0 + k * CHUNK, CHUNK)])
        plsc.subcore_barrier()

        def body(j, carry):
            pltpu.sync_copy(ones_v, acc.at[dst_v.at[j]], add=True)
            return carry

        lax.fori_loop(0, EPW_CHUNKS, body, 0)
        plsc.subcore_barrier()
        for k in range(ROWS_PER_TILE // CHUNK):
            pltpu.sync_copy(acc.at[pl.ds(r0 + k * CHUNK, CHUNK)], zb)
            pltpu.sync_copy(zb, degp.at[c, pl.ds(r0 + k * CHUNK, CHUNK)])

    @functools.partial(
        pl.kernel,
        out_type=jax.ShapeDtypeStruct((2, 2, NP, QD), jnp.float32),
        mesh=mesh,
        compiler_params=pltpu.CompilerParams(use_tc_tiling_on_sc=False),
        scratch_types=[
            pltpu.VMEM((2 * EPW_CHUNKS, CHUNK), jnp.int32),
            pltpu.VMEM((2 * EPW_CHUNKS, CHUNK), jnp.int32),
            pltpu.VMEM((4, CHUNK, QD), jnp.float32),
            pltpu.VMEM((CHUNK, QD), jnp.float32),
            pltpu.VMEM((CHUNK, QD), jnp.float32),
            pltpu.VMEM_SHARED((NP, QD), jnp.float32),
            pltpu.VMEM_SHARED((NP, QD), jnp.float32),
            pltpu.VMEM_SHARED((NP, QD), jnp.float32),
            pltpu.SemaphoreType.DMA,
            pltpu.SemaphoreType.DMA,
            pltpu.SemaphoreType.DMA,
            pltpu.SemaphoreType.DMA,
            pltpu.SemaphoreType.DMA,
            pltpu.SemaphoreType.DMA,
            pltpu.SemaphoreType.DMA,
            pltpu.SemaphoreType.DMA,
        ],
    )
    def prop_kernel(g, srcp, dstp, zeros32, parts,
                    src_v, dst_v, rowsr, zbz, zbs, tabA, tabB, acc,
                    gsem0, gsem1, gsem2, gsem3,
                    ssem0, ssem1, ssem2, ssem3):
        gsems = (gsem0, gsem1, gsem2, gsem3)
        ssems = (ssem0, ssem1, ssem2, ssem3)
        c = lax.axis_index("c")
        s = lax.axis_index("s")

        pltpu.sync_copy(srcp.at[2 * s], src_v.at[pl.ds(0, EPW_CHUNKS)])
        pltpu.sync_copy(srcp.at[2 * s + 1],
                        src_v.at[pl.ds(EPW_CHUNKS, EPW_CHUNKS)])
        pltpu.sync_copy(dstp.at[2 * s], dst_v.at[pl.ds(0, EPW_CHUNKS)])
        pltpu.sync_copy(dstp.at[2 * s + 1],
                        dst_v.at[pl.ds(EPW_CHUNKS, EPW_CHUNKS)])
        pltpu.sync_copy(zeros32, zbz)
        r0 = s * ROWS_PER_TILE

        @pl.when(c == 0)
        def _():
            pltpu.sync_copy(g.at[pl.ds(r0, ROWS_PER_TILE), pl.ds(0, QD)],
                            tabA.at[pl.ds(r0, ROWS_PER_TILE)])
            pltpu.sync_copy(g.at[pl.ds(r0, ROWS_PER_TILE), pl.ds(QD, QD)],
                            tabB.at[pl.ds(r0, ROWS_PER_TILE)])

        @pl.when(c == 1)
        def _():
            pltpu.sync_copy(
                g.at[pl.ds(r0, ROWS_PER_TILE), pl.ds(2 * QD, QD)],
                tabA.at[pl.ds(r0, ROWS_PER_TILE)])
            pltpu.sync_copy(
                g.at[pl.ds(r0, ROWS_PER_TILE), pl.ds(3 * QD, QD)],
                tabB.at[pl.ds(r0, ROWS_PER_TILE)])
        for k in range(ROWS_PER_TILE // CHUNK):
            pltpu.sync_copy(zbz, acc.at[pl.ds(r0 + k * CHUNK, CHUNK)])
        plsc.subcore_barrier()

        nch = 2 * EPW_CHUNKS
        NB = 4
        for p, tab in enumerate((tabA, tabB)):
            for b in range(NB - 1):
                pltpu.async_copy(tab.at[src_v.at[b]], rowsr.at[b],
                                 gsems[b])

            def body(i, carry):
                for b in range(NB):
                    jj = NB * i + b
                    nxt = (b + NB - 1) % NB

                    @pl.when((jj + NB - 1 < nch) & (jj > 0))
                    def _():
                        pltpu.make_async_copy(
                            rowsr.at[nxt], acc.at[dst_v.at[jj - 1]],
                            ssems[nxt]).wait()

                    @pl.when(jj + NB - 1 < nch)
                    def _():
                        pltpu.async_copy(tab.at[src_v.at[jj + NB - 1]],
                                         rowsr.at[nxt], gsems[nxt])

                    pltpu.make_async_copy(tab.at[src_v.at[jj]],
                                          rowsr.at[b], gsems[b]).wait()
                    pltpu.async_copy(rowsr.at[b], acc.at[dst_v.at[jj]],
                                     ssems[b], add=True)
                return carry

            lax.fori_loop(0, nch // NB, body, 0)
            for b in range(NB):
                pltpu.make_async_copy(rowsr.at[b],
                                      acc.at[dst_v.at[nch - NB + b]],
                                      ssems[b]).wait()
            plsc.subcore_barrier()
            for k in range(ROWS_PER_TILE // CHUNK):
                pltpu.sync_copy(acc.at[pl.ds(r0 + k * CHUNK, CHUNK)], zbs)
                pltpu.sync_copy(zbs,
                                parts.at[c, p, pl.ds(r0 + k * CHUNK, CHUNK)])
                if p == 0:
                    pltpu.sync_copy(zbz,
                                    acc.at[pl.ds(r0 + k * CHUNK, CHUNK)])
            if p == 0:
                plsc.subcore_barrier()

    return deg_kernel, prop_kernel


def _run_deg(dstp, ones16, zeros16):
    return _sc_kernels()[0](dstp, ones16, zeros16)


def _run_prop(g, srcp, dstp, zeros32):
    return _sc_kernels()[1](g, srcp, dstp, zeros32)


def _tc0_body(x_ref, w_ref, degp_ref, g_ref, dinv_ref):
    dp = degp_ref[...]
    deg = dp[0, :, 0:1] + dp[1, :, 0:1] + 1.0
    dinv = 1.0 / jnp.sqrt(deg)
    h = jnp.dot(x_ref[...], w_ref[...], preferred_element_type=jnp.float32)
    g_ref[...] = h * dinv
    dinv_ref[...] = dinv


def _tc0(x, w1, degp):
    return pl.pallas_call(
        _tc0_body,
        grid=(GRID,),
        in_specs=[
            pl.BlockSpec((BR, DD), lambda i: (i, 0)),
            pl.BlockSpec((DD, DD), lambda i: (0, 0)),
            pl.BlockSpec((2, BR, 16), lambda i: (0, i, 0)),
        ],
        out_specs=[
            pl.BlockSpec((BR, DD), lambda i: (i, 0)),
            pl.BlockSpec((BR, 1), lambda i: (i, 0)),
        ],
        out_shape=[
            jax.ShapeDtypeStruct((NP, DD), jnp.float32),
            jax.ShapeDtypeStruct((NN, 1), jnp.float32),
        ],
    )(x, w1, degp)


def _pcat(p):
    return jnp.concatenate([p[0, 0], p[0, 1], p[1, 0], p[1, 1]], axis=1)


def _tcmid_body(p_ref, g_ref, dinv_ref, b_ref, w_ref, out_ref):
    s = _pcat(p_ref[...])
    dinv = dinv_ref[...]
    xb = (s + g_ref[...]) * dinv + b_ref[...]
    xb = jnp.maximum(xb, 0.0)
    out_ref[...] = (
        jnp.dot(xb, w_ref[...], preferred_element_type=jnp.float32) * dinv
    )


def _tcmid(parts, gprev, dinv, b, wnext):
    return pl.pallas_call(
        _tcmid_body,
        grid=(GRID,),
        in_specs=[
            pl.BlockSpec((2, 2, BR, QD), lambda i: (0, 0, i, 0)),
            pl.BlockSpec((BR, DD), lambda i: (i, 0)),
            pl.BlockSpec((BR, 1), lambda i: (i, 0)),
            pl.BlockSpec((1, DD), lambda i: (0, 0)),
            pl.BlockSpec((DD, DD), lambda i: (0, 0)),
        ],
        out_specs=pl.BlockSpec((BR, DD), lambda i: (i, 0)),
        out_shape=jax.ShapeDtypeStruct((NP, DD), jnp.float32),
    )(parts, gprev, dinv, b, wnext)


def _tc3_body(p_ref, g_ref, dinv_ref, b_ref, batch_ref, wlin_ref, blin_ref,
              out_ref, acc_ref, cnt_ref):
    i = pl.program_id(0)

    @pl.when(i == 0)
    def _():
        acc_ref[...] = jnp.zeros_like(acc_ref)
        cnt_ref[...] = jnp.zeros_like(cnt_ref)

    s = _pcat(p_ref[...])
    h3 = (s + g_ref[...]) * dinv_ref[...] + b_ref[...]
    bidx = batch_ref[...]
    gids = lax.broadcasted_iota(jnp.int32, (1, GG), 1)
    mask = (bidx == gids).astype(jnp.float32)
    acc_ref[...] += lax.dot_general(
        mask, h3, (((0,), (0,)), ((), ())), preferred_element_type=jnp.float32
    )
    cnt_ref[...] += lax.dot_general(
        mask, jnp.ones((BR, 1), jnp.float32), (((0,), (0,)), ((), ())),
        preferred_element_type=jnp.float32,
    )

    @pl.when(i == pl.num_programs(0) - 1)
    def _():
        pooled = acc_ref[...] / jnp.maximum(cnt_ref[...], 1.0)
        out_ref[...] = (
            jnp.dot(pooled, wlin_ref[...], preferred_element_type=jnp.float32)
            + blin_ref[...]
        )


def _tc3(parts, g3, dinv, b3, batch2d, wlin, blin):
    return pl.pallas_call(
        _tc3_body,
        grid=(GRID,),
        in_specs=[
            pl.BlockSpec((2, 2, BR, QD), lambda i: (0, 0, i, 0)),
            pl.BlockSpec((BR, DD), lambda i: (i, 0)),
            pl.BlockSpec((BR, 1), lambda i: (i, 0)),
            pl.BlockSpec((1, DD), lambda i: (0, 0)),
            pl.BlockSpec((BR, 1), lambda i: (i, 0)),
            pl.BlockSpec((DD, 2), lambda i: (0, 0)),
            pl.BlockSpec((1, 2), lambda i: (0, 0)),
        ],
        out_specs=pl.BlockSpec((GG, 2), lambda i: (0, 0)),
        out_shape=jax.ShapeDtypeStruct((GG, 2), jnp.float32),
        scratch_shapes=[
            pltpu.VMEM((GG, DD), jnp.float32),
            pltpu.VMEM((GG, 1), jnp.float32),
        ],
    )(parts, g3, dinv, b3, batch2d, wlin, blin)


def kernel(x, edge_index, batch, W1, b1, W2, b2, W3, b3, Wlin, blin):
    src = edge_index[0].astype(jnp.int32)
    dst = edge_index[1].astype(jnp.int32)
    pad_e = EP - src.shape[0]
    srcp = jnp.concatenate([src, jnp.zeros((pad_e,), jnp.int32)])
    srcp = srcp.reshape(NW, EPW_CHUNKS, CHUNK)
    dstp = jnp.concatenate([dst, jnp.full((pad_e,), NP - 1, jnp.int32)])
    dstp = dstp.reshape(NW, EPW_CHUNKS, CHUNK)
    ones16 = jnp.ones((CHUNK, 16), jnp.float32)
    zeros16 = jnp.zeros((CHUNK, 16), jnp.float32)
    zeros32 = jnp.zeros((CHUNK, QD), jnp.float32)
    batch2d = batch.astype(jnp.int32).reshape(NN, 1)

    degp = _run_deg(dstp, ones16, zeros16)
    g1, dinv = _tc0(x, W1, degp)
    parts1 = _run_prop(g1, srcp, dstp, zeros32)
    g2 = _tcmid(parts1, g1, dinv, b1.reshape(1, DD), W2)
    parts2 = _run_prop(g2, srcp, dstp, zeros32)
    g3 = _tcmid(parts2, g2, dinv, b2.reshape(1, DD), W3)
    parts3 = _run_prop(g3, srcp, dstp, zeros32)
    return _tc3(parts3, g3, dinv, b3.reshape(1, DD), batch2d, Wlin,
                blin.reshape(1, 2))
